# baseline (speedup 1.0000x reference)
import functools

import jax
import jax.numpy as jnp
from jax import lax
from jax.experimental import pallas as pl
from jax.experimental.pallas import tpu as pltpu
from jax.experimental.pallas import tpu_sc as plsc

EMBED_DIM = 768
REGION_MAX = 116
NREG_PAD = 128
NW = 32
L = 16
ROUND_C = 12582912.0
BLK = 8192


def _inv4(m):
    a = [[m[i, j] for j in range(4)] for i in range(4)]
    (a00, a01, a02, a03), (a10, a11, a12, a13), \
        (a20, a21, a22, a23), (a30, a31, a32, a33) = a
    b00 = a00 * a11 - a01 * a10
    b01 = a00 * a12 - a02 * a10
    b02 = a00 * a13 - a03 * a10
    b03 = a01 * a12 - a02 * a11
    b04 = a01 * a13 - a03 * a11
    b05 = a02 * a13 - a03 * a12
    b06 = a20 * a31 - a21 * a30
    b07 = a20 * a32 - a22 * a30
    b08 = a20 * a33 - a23 * a30
    b09 = a21 * a32 - a22 * a31
    b10 = a21 * a33 - a23 * a31
    b11 = a22 * a33 - a23 * a32
    det = b00 * b11 - b01 * b10 + b02 * b09 + b03 * b08 - b04 * b07 + b05 * b06
    inv = jnp.stack([
        a11 * b11 - a12 * b10 + a13 * b09,
        a02 * b10 - a01 * b11 - a03 * b09,
        a31 * b05 - a32 * b04 + a33 * b03,
        a22 * b04 - a21 * b05 - a23 * b03,
        a12 * b08 - a10 * b11 - a13 * b07,
        a00 * b11 - a02 * b08 + a03 * b07,
        a32 * b02 - a30 * b05 - a33 * b01,
        a20 * b05 - a22 * b02 + a23 * b01,
        a10 * b10 - a11 * b08 + a13 * b06,
        a01 * b08 - a00 * b10 - a03 * b06,
        a30 * b04 - a31 * b02 + a33 * b00,
        a21 * b02 - a20 * b04 - a23 * b00,
        a11 * b07 - a10 * b09 - a12 * b06,
        a00 * b09 - a01 * b07 + a02 * b06,
        a31 * b01 - a30 * b03 - a32 * b00,
        a20 * b03 - a21 * b01 + a22 * b00,
    ]).reshape(4, 4)
    return inv / det


@functools.lru_cache(maxsize=None)
def _build_rid_kernel(D, H, W, n_pts):
    P = n_pts // NW
    GID = 128
    mesh = plsc.VectorSubcoreMesh(core_axis_name="c", subcore_axis_name="s")

    @functools.partial(
        pl.kernel,
        mesh=mesh,
        out_type=jax.ShapeDtypeStruct((n_pts,), jnp.int32),
        scratch_types=[
            pltpu.VMEM((P,), jnp.float32),
            pltpu.VMEM((P,), jnp.float32),
            pltpu.VMEM((P,), jnp.float32),
            pltpu.VMEM((P,), jnp.int32),
            pltpu.VMEM((P,), jnp.int32),
            pltpu.VMEM((P,), jnp.float32),
            pltpu.VMEM((P,), jnp.int32),
            pltpu.SemaphoreType.DMA,
        ],
    )
    def rid_kernel(coords_hbm, aal_hbm, rid_hbm,
                   x_v, y_v, z_v, idx_v, val_v, reg_v, rid_v, sem_a):
        wid = lax.axis_index("s") * 2 + lax.axis_index("c")
        base = wid * P

        pltpu.sync_copy(coords_hbm.at[pl.ds(base, P)], x_v)
        pltpu.sync_copy(coords_hbm.at[pl.ds(n_pts + base, P)], y_v)
        pltpu.sync_copy(coords_hbm.at[pl.ds(2 * n_pts + base, P)], z_v)

        def coord_body(i, carry):
            sl = pl.ds(pl.multiple_of(i * L, L), L)
            fx = (x_v[sl] + ROUND_C) - ROUND_C
            fy = (y_v[sl] + ROUND_C) - ROUND_C
            fz = (z_v[sl] + ROUND_C) - ROUND_C
            xi = fx.astype(jnp.int32)
            yi = fy.astype(jnp.int32)
            zi = fz.astype(jnp.int32)
            valid = ((xi >= 0) & (xi < D) & (yi >= 0) & (yi < H)
                     & (zi >= 0) & (zi < W))
            xc = jnp.minimum(jnp.maximum(xi, 0), D - 1)
            yc = jnp.minimum(jnp.maximum(yi, 0), H - 1)
            zc = jnp.minimum(jnp.maximum(zi, 0), W - 1)
            idx_v[sl] = (xc * (H * W) + yc * W + zc)
            val_v[sl] = jnp.where(valid, 1, 0)
            return carry

        lax.fori_loop(0, P // L, coord_body, 0)

        atlas_copies = []
        for j in range(P // GID):
            sl = pl.ds(j * GID, GID)
            atlas_copies.append(
                pltpu.async_copy(aal_hbm.at[idx_v.at[sl]], reg_v.at[sl], sem_a))
        for cp in atlas_copies:
            cp.wait()

        def region_body(i, carry):
            sl = pl.ds(pl.multiple_of(i * L, L), L)
            r = reg_v[sl].astype(jnp.int32)
            ok = (r >= 0) & (r <= REGION_MAX) & (val_v[sl] > 0)
            rid_v[sl] = jnp.where(ok, r, 0)
            return carry

        lax.fori_loop(0, P // L, region_body, 0)
        pltpu.sync_copy(rid_v, rid_hbm.at[pl.ds(base, P)])

    return rid_kernel


def _expand_body(rid_ref, tab_ref, out_ref):
    rid = rid_ref[0, 0, :]
    onehot = (rid[:, None]
              == lax.broadcasted_iota(jnp.int32, (BLK, NREG_PAD), 1))
    out_ref[...] = jnp.dot(onehot.astype(jnp.float32), tab_ref[...],
                           preferred_element_type=jnp.float32)


@functools.lru_cache(maxsize=None)
def _build_expand(n_pts):
    grid = n_pts // BLK
    return pl.pallas_call(
        _expand_body,
        grid=(grid,),
        in_specs=[
            pl.BlockSpec((1, 1, BLK), lambda i: (i, 0, 0)),
            pl.BlockSpec((NREG_PAD, EMBED_DIM), lambda i: (0, 0)),
        ],
        out_specs=pl.BlockSpec((BLK, EMBED_DIM), lambda i: (i, 0)),
        out_shape=jax.ShapeDtypeStruct((n_pts, EMBED_DIM), jnp.float32),
    )


def kernel(patch_centers_voxels, mri_affine, aal_affine, aal_data, region_embed):
    Bb, Nn, _ = patch_centers_voxels.shape
    D, H, W = aal_data.shape
    n_pts = Bb * Nn
    ones = jnp.ones((Bb, Nn, 1), dtype=jnp.float32)
    voxel_homo = jnp.concatenate(
        [patch_centers_voxels.astype(jnp.float32), ones], axis=-1)
    world_coords = jnp.einsum('ij,bnj->bni', mri_affine, voxel_homo)
    inv_aal_affine = jnp.linalg.inv(aal_affine)
    aal_voxel_coords = jnp.einsum('ij,bnj->bni', inv_aal_affine, world_coords)
    coords_t = aal_voxel_coords[..., :3].reshape(n_pts, 3).T.reshape(-1)
    aal_flat = aal_data.astype(jnp.float32).reshape(-1)
    rid = _build_rid_kernel(D, H, W, n_pts)(coords_t, aal_flat)
    tab = jnp.zeros((NREG_PAD, EMBED_DIM), jnp.float32).at[:REGION_MAX + 1].set(
        region_embed.astype(jnp.float32))
    out = _build_expand(n_pts)(rid.reshape(n_pts // BLK, 1, BLK), tab)
    return out.reshape(Bb, Nn, EMBED_DIM)


# speedup vs baseline: 1.0821x; 1.0821x over previous
import functools

import jax
import jax.numpy as jnp
from jax import lax
from jax.experimental import pallas as pl
from jax.experimental.pallas import tpu as pltpu
from jax.experimental.pallas import tpu_sc as plsc

EMBED_DIM = 768
REGION_MAX = 116
NREG_PAD = 128
NW = 32
L = 16
ROUND_C = 12582912.0
BLK = 2048


def _inv4(m):
    a = [[m[i, j] for j in range(4)] for i in range(4)]
    (a00, a01, a02, a03), (a10, a11, a12, a13), \
        (a20, a21, a22, a23), (a30, a31, a32, a33) = a
    b00 = a00 * a11 - a01 * a10
    b01 = a00 * a12 - a02 * a10
    b02 = a00 * a13 - a03 * a10
    b03 = a01 * a12 - a02 * a11
    b04 = a01 * a13 - a03 * a11
    b05 = a02 * a13 - a03 * a12
    b06 = a20 * a31 - a21 * a30
    b07 = a20 * a32 - a22 * a30
    b08 = a20 * a33 - a23 * a30
    b09 = a21 * a32 - a22 * a31
    b10 = a21 * a33 - a23 * a31
    b11 = a22 * a33 - a23 * a32
    det = b00 * b11 - b01 * b10 + b02 * b09 + b03 * b08 - b04 * b07 + b05 * b06
    inv = jnp.stack([
        a11 * b11 - a12 * b10 + a13 * b09,
        a02 * b10 - a01 * b11 - a03 * b09,
        a31 * b05 - a32 * b04 + a33 * b03,
        a22 * b04 - a21 * b05 - a23 * b03,
        a12 * b08 - a10 * b11 - a13 * b07,
        a00 * b11 - a02 * b08 + a03 * b07,
        a32 * b02 - a30 * b05 - a33 * b01,
        a20 * b05 - a22 * b02 + a23 * b01,
        a10 * b10 - a11 * b08 + a13 * b06,
        a01 * b08 - a00 * b10 - a03 * b06,
        a30 * b04 - a31 * b02 + a33 * b00,
        a21 * b02 - a20 * b04 - a23 * b00,
        a11 * b07 - a10 * b09 - a12 * b06,
        a00 * b09 - a01 * b07 + a02 * b06,
        a31 * b01 - a30 * b03 - a32 * b00,
        a20 * b03 - a21 * b01 + a22 * b00,
    ]).reshape(4, 4)
    return inv / det


@functools.lru_cache(maxsize=None)
def _build_rid_kernel(D, H, W, n_pts):
    P = n_pts // NW
    GID = 128
    mesh = plsc.VectorSubcoreMesh(core_axis_name="c", subcore_axis_name="s")

    @functools.partial(
        pl.kernel,
        mesh=mesh,
        out_type=jax.ShapeDtypeStruct((n_pts,), jnp.int32),
        scratch_types=[
            pltpu.VMEM((P,), jnp.float32),
            pltpu.VMEM((P,), jnp.float32),
            pltpu.VMEM((P,), jnp.float32),
            pltpu.VMEM((P,), jnp.int32),
            pltpu.VMEM((P,), jnp.int32),
            pltpu.VMEM((P,), jnp.float32),
            pltpu.VMEM((P,), jnp.int32),
            pltpu.SemaphoreType.DMA,
        ],
    )
    def rid_kernel(coords_hbm, aal_hbm, rid_hbm,
                   x_v, y_v, z_v, idx_v, val_v, reg_v, rid_v, sem_a):
        wid = lax.axis_index("s") * 2 + lax.axis_index("c")
        base = wid * P

        in_copies = [
            pltpu.async_copy(coords_hbm.at[pl.ds(base, P)], x_v, sem_a),
            pltpu.async_copy(coords_hbm.at[pl.ds(n_pts + base, P)], y_v, sem_a),
            pltpu.async_copy(coords_hbm.at[pl.ds(2 * n_pts + base, P)], z_v,
                             sem_a),
        ]
        for cp in in_copies:
            cp.wait()

        def coord_body(i, carry):
            sl = pl.ds(pl.multiple_of(i * L, L), L)
            fx = (x_v[sl] + ROUND_C) - ROUND_C
            fy = (y_v[sl] + ROUND_C) - ROUND_C
            fz = (z_v[sl] + ROUND_C) - ROUND_C
            xi = fx.astype(jnp.int32)
            yi = fy.astype(jnp.int32)
            zi = fz.astype(jnp.int32)
            valid = ((xi >= 0) & (xi < D) & (yi >= 0) & (yi < H)
                     & (zi >= 0) & (zi < W))
            xc = jnp.minimum(jnp.maximum(xi, 0), D - 1)
            yc = jnp.minimum(jnp.maximum(yi, 0), H - 1)
            zc = jnp.minimum(jnp.maximum(zi, 0), W - 1)
            idx_v[sl] = (xc * (H * W) + yc * W + zc)
            val_v[sl] = jnp.where(valid, 1, 0)
            return carry

        lax.fori_loop(0, P // L, coord_body, 0)

        atlas_copies = []
        for j in range(P // GID):
            sl = pl.ds(j * GID, GID)
            atlas_copies.append(
                pltpu.async_copy(aal_hbm.at[idx_v.at[sl]], reg_v.at[sl], sem_a))
        for cp in atlas_copies:
            cp.wait()

        def region_body(i, carry):
            sl = pl.ds(pl.multiple_of(i * L, L), L)
            r = reg_v[sl].astype(jnp.int32)
            ok = (r >= 0) & (r <= REGION_MAX) & (val_v[sl] > 0)
            rid_v[sl] = jnp.where(ok, r, 0)
            return carry

        lax.fori_loop(0, P // L, region_body, 0)
        pltpu.sync_copy(rid_v, rid_hbm.at[pl.ds(base, P)])

    return rid_kernel


def _expand_body(rid_ref, tab_ref, out_ref):
    rid = rid_ref[0, 0, :]
    onehot = (rid[:, None]
              == lax.broadcasted_iota(jnp.int32, (BLK, NREG_PAD), 1))
    out_ref[...] = jnp.dot(onehot.astype(jnp.float32), tab_ref[...],
                           preferred_element_type=jnp.float32)


@functools.lru_cache(maxsize=None)
def _build_expand(n_pts):
    grid = n_pts // BLK
    return pl.pallas_call(
        _expand_body,
        grid=(grid,),
        in_specs=[
            pl.BlockSpec((1, 1, BLK), lambda i: (i, 0, 0)),
            pl.BlockSpec((NREG_PAD, EMBED_DIM), lambda i: (0, 0)),
        ],
        out_specs=pl.BlockSpec((BLK, EMBED_DIM), lambda i: (i, 0)),
        out_shape=jax.ShapeDtypeStruct((n_pts, EMBED_DIM), jnp.float32),
    )


def kernel(patch_centers_voxels, mri_affine, aal_affine, aal_data, region_embed):
    Bb, Nn, _ = patch_centers_voxels.shape
    D, H, W = aal_data.shape
    n_pts = Bb * Nn
    ones = jnp.ones((Bb, Nn, 1), dtype=jnp.float32)
    voxel_homo = jnp.concatenate(
        [patch_centers_voxels.astype(jnp.float32), ones], axis=-1)
    world_coords = jnp.einsum('ij,bnj->bni', mri_affine, voxel_homo)
    inv_aal_affine = jnp.linalg.inv(aal_affine)
    aal_voxel_coords = jnp.einsum('ij,bnj->bni', inv_aal_affine, world_coords)
    coords_t = aal_voxel_coords[..., :3].reshape(n_pts, 3).T.reshape(-1)
    aal_flat = aal_data.astype(jnp.float32).reshape(-1)
    rid = _build_rid_kernel(D, H, W, n_pts)(coords_t, aal_flat)
    tab = jnp.zeros((NREG_PAD, EMBED_DIM), jnp.float32).at[:REGION_MAX + 1].set(
        region_embed.astype(jnp.float32))
    out = _build_expand(n_pts)(rid.reshape(n_pts // BLK, 1, BLK), tab)
    return out.reshape(Bb, Nn, EMBED_DIM)


# speedup vs baseline: 1.0833x; 1.0011x over previous
import functools

import jax
import jax.numpy as jnp
from jax import lax
from jax.experimental import pallas as pl
from jax.experimental.pallas import tpu as pltpu
from jax.experimental.pallas import tpu_sc as plsc

EMBED_DIM = 768
REGION_MAX = 116
NREG_PAD = 128
NW = 32
L = 16
ROUND_C = 12582912.0
BLK = 2048


@functools.lru_cache(maxsize=None)
def _build_rid_kernel(D, H, W, n_pts):
    P = n_pts // NW
    GID = 128
    mesh = plsc.VectorSubcoreMesh(core_axis_name="c", subcore_axis_name="s")

    @functools.partial(
        pl.kernel,
        mesh=mesh,
        out_type=jax.ShapeDtypeStruct((n_pts,), jnp.int32),
        scratch_types=[
            pltpu.VMEM((P,), jnp.float32),
            pltpu.VMEM((P,), jnp.float32),
            pltpu.VMEM((P,), jnp.float32),
            pltpu.VMEM((P,), jnp.int32),
            pltpu.VMEM((P,), jnp.int32),
            pltpu.VMEM((P,), jnp.float32),
            pltpu.VMEM((P,), jnp.int32),
            pltpu.SemaphoreType.DMA,
        ],
    )
    def rid_kernel(coords_hbm, aal_hbm, rid_hbm,
                   x_v, y_v, z_v, idx_v, val_v, reg_v, rid_v, sem_a):
        wid = lax.axis_index("s") * 2 + lax.axis_index("c")
        base = wid * P

        in_copies = [
            pltpu.async_copy(coords_hbm.at[pl.ds(base, P)], x_v, sem_a),
            pltpu.async_copy(coords_hbm.at[pl.ds(n_pts + base, P)], y_v, sem_a),
            pltpu.async_copy(coords_hbm.at[pl.ds(2 * n_pts + base, P)], z_v,
                             sem_a),
        ]
        for cp in in_copies:
            cp.wait()

        def coord_body(i, carry):
            sl = pl.ds(pl.multiple_of(i * L, L), L)
            fx = (x_v[sl] + ROUND_C) - ROUND_C
            fy = (y_v[sl] + ROUND_C) - ROUND_C
            fz = (z_v[sl] + ROUND_C) - ROUND_C
            xi = fx.astype(jnp.int32)
            yi = fy.astype(jnp.int32)
            zi = fz.astype(jnp.int32)
            valid = ((xi >= 0) & (xi < D) & (yi >= 0) & (yi < H)
                     & (zi >= 0) & (zi < W))
            xc = jnp.minimum(jnp.maximum(xi, 0), D - 1)
            yc = jnp.minimum(jnp.maximum(yi, 0), H - 1)
            zc = jnp.minimum(jnp.maximum(zi, 0), W - 1)
            idx_v[sl] = (xc * (H * W) + yc * W + zc)
            val_v[sl] = jnp.where(valid, 1, 0)
            return carry

        lax.fori_loop(0, P // L, coord_body, 0)

        atlas_copies = []
        for j in range(P // GID):
            sl = pl.ds(j * GID, GID)
            atlas_copies.append(
                pltpu.async_copy(aal_hbm.at[idx_v.at[sl]], reg_v.at[sl], sem_a))
        for cp in atlas_copies:
            cp.wait()

        def region_body(i, carry):
            sl = pl.ds(pl.multiple_of(i * L, L), L)
            r = reg_v[sl].astype(jnp.int32)
            ok = (r >= 0) & (r <= REGION_MAX) & (val_v[sl] > 0)
            rid_v[sl] = jnp.where(ok, r, 0)
            return carry

        lax.fori_loop(0, P // L, region_body, 0)
        pltpu.sync_copy(rid_v, rid_hbm.at[pl.ds(base, P)])

    return rid_kernel


def _expand_body(rid_ref, tab_ref, out_ref):
    rid = rid_ref[0, 0, :]
    onehot = (rid[:, None]
              == lax.broadcasted_iota(jnp.int32, (BLK, NREG_PAD), 1))
    out_ref[...] = jnp.dot(onehot.astype(jnp.float32), tab_ref[...],
                           preferred_element_type=jnp.float32)


@functools.lru_cache(maxsize=None)
def _build_expand(n_pts):
    grid = n_pts // BLK
    return pl.pallas_call(
        _expand_body,
        grid=(grid,),
        in_specs=[
            pl.BlockSpec((1, 1, BLK), lambda i: (i, 0, 0)),
            pl.BlockSpec((NREG_PAD, EMBED_DIM), lambda i: (0, 0)),
        ],
        out_specs=pl.BlockSpec((BLK, EMBED_DIM), lambda i: (i, 0)),
        out_shape=jax.ShapeDtypeStruct((n_pts, EMBED_DIM), jnp.float32),
    )


def kernel(patch_centers_voxels, mri_affine, aal_affine, aal_data, region_embed):
    Bb, Nn, _ = patch_centers_voxels.shape
    D, H, W = aal_data.shape
    n_pts = Bb * Nn
    ones = jnp.ones((Bb, Nn, 1), dtype=jnp.float32)
    voxel_homo = jnp.concatenate(
        [patch_centers_voxels.astype(jnp.float32), ones], axis=-1)
    world_coords = jnp.einsum('ij,bnj->bni', mri_affine, voxel_homo)
    inv_aal_affine = jnp.linalg.inv(aal_affine)
    aal_voxel_coords = jnp.einsum('ij,bnj->bni', inv_aal_affine, world_coords)
    coords_t = aal_voxel_coords[..., :3].reshape(n_pts, 3).T.reshape(-1)
    aal_flat = aal_data.astype(jnp.float32).reshape(-1)
    rid = _build_rid_kernel(D, H, W, n_pts)(coords_t, aal_flat)
    tab = jnp.zeros((NREG_PAD, EMBED_DIM), jnp.float32).at[:REGION_MAX + 1].set(
        region_embed.astype(jnp.float32))
    out = _build_expand(n_pts)(rid.reshape(n_pts // BLK, 1, BLK), tab)
    return out.reshape(Bb, Nn, EMBED_DIM)
